# Initial kernel scaffold; baseline (speedup 1.0000x reference)
import jax
import jax.numpy as jnp
from jax.experimental import pallas as pl


def kernel(z, codebook):
    raise NotImplementedError("write your pallas kernel here")


# speedup vs baseline: 1.5102x; 1.5102x over previous
import functools

import jax
import jax.numpy as jnp
from jax import lax
from jax.experimental import pallas as pl
from jax.experimental.pallas import tpu as pltpu
from jax.experimental.pallas import tpu_sc as plsc

_K = 8192
_D = 64
_BETA = 0.25
_N = 18432

_ROWS = 512
_KCHUNK = 1024
_NT = _N // _ROWS
_NJ = _K // _KCHUNK
_NV = _KCHUNK // 128


def _argmin_body(x_ref, cb_ref, x2_ref, c2_ref, idx_ref, loss_ref):
    i = pl.program_id(0)
    x = x_ref[...]
    xm2 = (x * (-2.0)).astype(jnp.bfloat16)
    x2 = x2_ref[...]

    runmin = jnp.full((_ROWS, 128), jnp.inf, dtype=jnp.float32)
    runc = jnp.zeros((_ROWS, 128), dtype=jnp.int32)
    for j in range(_NJ):
        c = cb_ref[pl.ds(j * _KCHUNK, _KCHUNK), :]
        d = lax.dot_general(xm2, c, (((1,), (1,)), ((), ())),
                            preferred_element_type=jnp.float32)
        s = (x2 + d) + c2_ref[:, pl.ds(j * _KCHUNK, _KCHUNK)]
        for v in range(_NV):
            sv = lax.slice(s, (0, v * 128), (_ROWS, v * 128 + 128))
            upd = sv < runmin
            runmin = jnp.where(upd, sv, runmin)
            runc = jnp.where(upd, j * _NV + v, runc)

    rowmin = jnp.min(runmin, axis=1, keepdims=True)
    lane = lax.broadcasted_iota(jnp.int32, (_ROWS, 128), 1)
    cand = runc * 128 + lane
    sel = jnp.where(runmin == rowmin, cand, jnp.int32(2**31 - 1))
    idx = jnp.min(sel, axis=1)
    idx_ref[...] = idx.reshape(1, 1, _ROWS)

    @pl.when(i == 0)
    def _init():
        loss_ref[...] = jnp.zeros_like(loss_ref)

    loss_ref[...] += jnp.sum(rowmin).reshape(1, 1)


_argmin_call = pl.pallas_call(
    _argmin_body,
    grid=(_NT,),
    in_specs=[
        pl.BlockSpec((_ROWS, _D), lambda i: (i, 0)),
        pl.BlockSpec((_K, _D), lambda i: (0, 0)),
        pl.BlockSpec((_ROWS, 1), lambda i: (i, 0)),
        pl.BlockSpec((1, _K), lambda i: (0, 0)),
    ],
    out_specs=[
        pl.BlockSpec((1, 1, _ROWS), lambda i: (i, 0, 0)),
        pl.BlockSpec((1, 1), lambda i: (0, 0)),
    ],
    out_shape=[
        jax.ShapeDtypeStruct((_NT, 1, _ROWS), jnp.int32),
        jax.ShapeDtypeStruct((1, 1), jnp.float32),
    ],
    compiler_params=pltpu.CompilerParams(
        dimension_semantics=("arbitrary",),
    ),
)


_NCORES = 2
_NSUB = 16
_NWORKERS = _NCORES * _NSUB
_BPW = _N // _NWORKERS


@functools.cache
def _make_sc_gather():
    @functools.partial(
        pl.kernel,
        mesh=plsc.VectorSubcoreMesh(core_axis_name="c", subcore_axis_name="s"),
        out_type=jax.ShapeDtypeStruct((_N, _D), jnp.float32),
        scratch_types=[
            pltpu.VMEM((_BPW,), jnp.int32),
            pltpu.VMEM((_BPW, _D), jnp.float32),
            pltpu.SemaphoreType.DMA,
        ],
        compiler_params=pltpu.CompilerParams(use_tc_tiling_on_sc=False),
    )
    def _sc_gather(table_hbm, idx_hbm, out_hbm, idx_v, rows_v, sem):
        wid = lax.axis_index("s") * _NCORES + lax.axis_index("c")
        base = wid * _BPW
        pltpu.sync_copy(idx_hbm.at[pl.ds(base, _BPW)], idx_v)
        pltpu.async_copy(table_hbm.at[idx_v], rows_v, sem).wait()
        pltpu.sync_copy(rows_v, out_hbm.at[pl.ds(base, _BPW)])

    return _sc_gather


def kernel(z, codebook):
    flat = z.reshape(-1, _D)
    x2 = jnp.sum(flat ** 2, axis=1, keepdims=True)
    c2 = jnp.sum(codebook ** 2, axis=1)
    idx3, loss_sum = _argmin_call(flat, codebook, x2, c2.reshape(1, _K))
    indices = idx3.reshape(z.shape[:-1])
    quantized = _make_sc_gather()(codebook, idx3.reshape(-1))
    quantized_st = quantized.reshape(z.shape)
    loss = _BETA * loss_sum[0, 0] / (_N * _D)
    return quantized_st, loss, indices
